# baseline (speedup 1.0000x reference)
import functools

import jax
import jax.numpy as jnp
from jax import lax
from jax.experimental import pallas as pl
from jax.experimental.pallas import tpu as pltpu
from jax.experimental.pallas import tpu_sc as plsc

B = 1024
D = 64
N_CAND = 100000
K = 100
S = 128
CHUNK = 4096
N_PAD = 102400
N_CHUNKS = N_PAD // CHUNK
G = N_PAD // S
GPC = CHUNK // S
GV = G // 16

NEG = float(jnp.finfo(jnp.float32).min)

NUM_CORES = 2
NUM_SUBCORES = 16
NW = NUM_CORES * NUM_SUBCORES
ROWS_PW = B // NW
GID_CAP = 128
SV = GID_CAP * S // 16
SURV_CAP = 256
SKV = SURV_CAP // 16
OUT_W = 128


def _scores_body(qemb_ref, wq_ref, cand_ref, wc_ref, scores_ref, gmax_ref):
    i = pl.program_id(0)
    q = jnp.dot(qemb_ref[...], wq_ref[...], preferred_element_type=jnp.float32)
    c = jnp.dot(cand_ref[...], wc_ref[...], preferred_element_type=jnp.float32)
    s = lax.dot_general(q, c, (((1,), (1,)), ((), ())),
                        preferred_element_type=jnp.float32)
    col = i * CHUNK + lax.broadcasted_iota(jnp.int32, (B, CHUNK), 1)
    s = jnp.where(col < N_CAND, s, NEG)
    scores_ref[...] = s
    gmax_ref[...] = jnp.max(s.reshape(B, GPC, S), axis=2)[None]


def _compute_scores(q_emb, Wq, cand_p, Wc):
    return pl.pallas_call(
        _scores_body,
        grid=(N_CHUNKS,),
        in_specs=[
            pl.BlockSpec((B, D), lambda i: (0, 0)),
            pl.BlockSpec((D, D), lambda i: (0, 0)),
            pl.BlockSpec((CHUNK, D), lambda i: (i, 0)),
            pl.BlockSpec((D, D), lambda i: (0, 0)),
        ],
        out_specs=[
            pl.BlockSpec((B, CHUNK), lambda i: (0, i)),
            pl.BlockSpec((1, B, GPC), lambda i: (i, 0, 0)),
        ],
        out_shape=[
            jax.ShapeDtypeStruct((B, N_PAD), jnp.float32),
            jax.ShapeDtypeStruct((N_CHUNKS, B, GPC), jnp.float32),
        ],
    )(q_emb, Wq, cand_p, Wc)


def _key_from_f32(x):
    b = lax.bitcast_convert_type(x, jnp.uint32)
    k = b ^ ((b >> 31) * jnp.uint32(0x7FFFFFFF))
    return lax.bitcast_convert_type(k, jnp.int32)


def _f32_from_key(ki):
    u = lax.bitcast_convert_type(ki, jnp.uint32)
    b = u ^ ((u >> 31) * jnp.uint32(0x7FFFFFFF))
    return lax.bitcast_convert_type(b, jnp.float32)


def _iota16():
    return lax.iota(jnp.int32, 16)


I32_MIN = -2147483648


def _kth_largest(keys_ref, nvec, k_want, nbits, stop_count=None, unroll2=False):
    def count_ge(cand_i):
        if unroll2:
            def cnt_body(j, acc):
                kv0 = keys_ref[pl.ds(j * 32, 16)]
                kv1 = keys_ref[pl.ds(j * 32 + 16, 16)]
                return (acc + (kv0 >= cand_i).astype(jnp.int32)
                        + (kv1 >= cand_i).astype(jnp.int32))
            acc = lax.fori_loop(0, nvec // 2, cnt_body,
                                jnp.zeros((16,), jnp.int32))
        else:
            def cnt_body(j, acc):
                kv = keys_ref[pl.ds(j * 16, 16)]
                return acc + (kv >= cand_i).astype(jnp.int32)
            acc = lax.fori_loop(0, nvec, cnt_body, jnp.zeros((16,), jnp.int32))
        return jnp.sum(acc)

    def step(b, prefix_u):
        shift = (31 - b).astype(jnp.uint32)
        cand_u = prefix_u | (jnp.uint32(1) << shift)
        cand_i = lax.bitcast_convert_type(cand_u ^ jnp.uint32(0x80000000),
                                          jnp.int32)
        cnt = count_ge(cand_i)
        return jnp.where(cnt >= k_want, cand_u, prefix_u), cnt

    if stop_count is None:
        def bit_body(b, prefix_u):
            return step(b, prefix_u)[0]
        prefix_u = lax.fori_loop(0, nbits, bit_body, jnp.uint32(0))
    else:
        def w_cond(st):
            b, _, best_cnt = st
            return (b < nbits) & (best_cnt > stop_count)

        def w_body(st):
            b, prefix_u, best_cnt = st
            new_prefix, cnt = step(b, prefix_u)
            best_cnt = jnp.where(cnt >= k_want, cnt, best_cnt)
            return b + 1, new_prefix, best_cnt

        _, prefix_u, _ = lax.while_loop(
            w_cond, w_body, (jnp.int32(0), jnp.uint32(0), jnp.int32(2**30)))
    return lax.bitcast_convert_type(prefix_u ^ jnp.uint32(0x80000000), jnp.int32)


def _merge2(ka, va, kb, vb):
    kbr = lax.rev(kb, (0,))
    vbr = lax.rev(vb, (0,))
    m = ka >= kbr
    hk = jnp.where(m, ka, kbr)
    hv = jnp.where(m, va, vbr)
    lk = jnp.where(m, kbr, ka)
    lv = jnp.where(m, vbr, va)
    hk, hv = plsc.sort_key_val(hk, hv, descending=True)
    lk, lv = plsc.sort_key_val(lk, lv, descending=True)
    return hk, hv, lk, lv


_NET8 = [(0, 1), (2, 3), (4, 5), (6, 7),
         (0, 2), (1, 3), (4, 6), (5, 7),
         (1, 2), (5, 6),
         (0, 4), (1, 5), (2, 6), (3, 7),
         (2, 4), (3, 5),
         (1, 2), (3, 4), (5, 6)]


def _sc_mesh():
    return plsc.VectorSubcoreMesh(core_axis_name="c", subcore_axis_name="s",
                                  num_cores=NUM_CORES, num_subcores=NUM_SUBCORES)


def _query_gather(query_table, query_ids):
    bpw = B // NW

    @functools.partial(
        pl.kernel, mesh=_sc_mesh(),
        compiler_params=pltpu.CompilerParams(needs_layout_passes=False),
        out_type=jax.ShapeDtypeStruct((B, 128), jnp.float32),
        scratch_types=[pltpu.VMEM((bpw,), jnp.int32),
                       pltpu.VMEM((bpw, 128), jnp.float32),
                       pltpu.SemaphoreType.DMA],
    )
    def kern(table_hbm, ids_hbm, out_hbm, idx_v, rows_v, sem):
        wid = lax.axis_index("s") * NUM_CORES + lax.axis_index("c")
        base = wid * bpw
        pltpu.sync_copy(ids_hbm.at[pl.ds(base, bpw)], idx_v)
        pltpu.async_copy(table_hbm.at[idx_v], rows_v, sem).wait()
        pltpu.sync_copy(rows_v, out_hbm.at[pl.ds(base, bpw)])

    table_p = jnp.pad(query_table, ((0, 0), (0, 128 - D)))
    return kern(table_p, query_ids)[:, :D]


def _sc_topk(gmax, scores2d):
    @functools.partial(
        pl.kernel, mesh=_sc_mesh(),
        compiler_params=pltpu.CompilerParams(needs_layout_passes=False),
        out_type=(jax.ShapeDtypeStruct((B, OUT_W), jnp.float32),
                  jax.ShapeDtypeStruct((B, OUT_W), jnp.int32)),
        scratch_types=[
            pltpu.VMEM((G,), jnp.float32),
            pltpu.VMEM((G,), jnp.int32),
            pltpu.VMEM((GID_CAP + 16,), jnp.int32),
            pltpu.VMEM((GID_CAP,), jnp.int32),
            pltpu.VMEM((GID_CAP, S), jnp.float32),
            pltpu.VMEM((SURV_CAP + 16,), jnp.int32),
            pltpu.VMEM((SURV_CAP + 16,), jnp.int32),
            pltpu.VMEM((OUT_W,), jnp.int32),
            pltpu.VMEM((OUT_W,), jnp.int32),
            pltpu.VMEM((OUT_W,), jnp.float32),
            pltpu.SemaphoreType.DMA,
        ],
    )
    def kern(gmax_hbm, sc2_hbm, outv_hbm, outi_hbm,
             gm_v, gk_v, gid_v, idx_v, gat_v, sk_v, si_v, tk_v, ti_v, tf_v, sem):
        wid = lax.axis_index("s") * NUM_CORES + lax.axis_index("c")
        iota = _iota16()
        zero16 = jnp.zeros((16,), jnp.int32)
        pad16 = jnp.full((16,), I32_MIN, jnp.int32)

        def row_body(rr, _carry):
            row = wid * ROWS_PW + rr

            pltpu.sync_copy(gmax_hbm.at[row], gm_v)

            def keyb(j, c):
                gk_v[pl.ds(j * 16, 16)] = _key_from_f32(gm_v[pl.ds(j * 16, 16)])
                return c
            lax.fori_loop(0, GV, keyb, 0)

            t_g = _kth_largest(gk_v, GV, K, 24,
                               stop_count=GID_CAP, unroll2=True)

            for qq in range(GID_CAP // 16):
                gid_v[pl.ds(qq * 16, 16)] = zero16

            def gsel(j, cnt):
                kv = gk_v[pl.ds(j * 16, 16)]
                ind = kv >= t_g
                cw = jnp.minimum(cnt, GID_CAP)
                plsc.store_compressed(gid_v.at[pl.ds(cw, 16)], j * 16 + iota,
                                      mask=ind)
                return cnt + jnp.sum(ind.astype(jnp.int32))
            cnt_g = jnp.minimum(lax.fori_loop(0, GV, gsel, 0), GID_CAP)

            for qq in range(GID_CAP // 16):
                idx_v[pl.ds(qq * 16, 16)] = gid_v[pl.ds(qq * 16, 16)] + row * G
            pltpu.async_copy(sc2_hbm.at[idx_v], gat_v, sem).wait()

            neg1_16 = zero16 - 1
            for qq in range(SKV):
                sk_v[pl.ds(qq * 16, 16)] = neg1_16
                si_v[pl.ds(qq * 16, 16)] = zero16

            t_gf = _f32_from_key(t_g)

            def fsel(s_slot, scnt):
                gidv = plsc.load_gather(gid_v, [zero16 + s_slot])
                base = gidv * S
                xs = [gat_v[s_slot, pl.ds(q * 16, 16)] for q in range(S // 16)]
                inds = [x >= t_gf for x in xs]
                cnts = [jnp.sum(i.astype(jnp.int32)) for i in inds]
                c = scnt
                for q in range(S // 16):
                    cw = jnp.minimum(c, SURV_CAP)
                    plsc.store_compressed(
                        sk_v.at[pl.ds(cw, 16)],
                        lax.bitcast_convert_type(xs[q], jnp.int32),
                        mask=inds[q])
                    plsc.store_compressed(si_v.at[pl.ds(cw, 16)],
                                          base + q * 16 + iota, mask=inds[q])
                    c = c + cnts[q]
                return c
            scnt = lax.fori_loop(0, cnt_g, fsel, 0)
            scnt_c = jnp.minimum(scnt, SURV_CAP)
            nsv = jnp.minimum((scnt_c + 31) // 32 * 2, SKV)

            def keypass(j, c):
                sk_v[pl.ds(j * 16, 16)] = _key_from_f32(sk_v[pl.ds(j * 16, 16)])
                return c
            lax.fori_loop(0, nsv, keypass, 0)

            t_e = _kth_largest(sk_v, nsv, K, 32, unroll2=True)

            for qq in range(OUT_W // 16):
                tk_v[pl.ds(qq * 16, 16)] = pad16
                ti_v[pl.ds(qq * 16, 16)] = zero16

            def tsel(j, cnt, strict):
                kv = sk_v[pl.ds(j * 16, 16)]
                iv = si_v[pl.ds(j * 16, 16)]
                ind = (kv > t_e) if strict else (kv == t_e)
                cw = jnp.minimum(cnt, K)
                plsc.store_compressed(tk_v.at[pl.ds(cw, 16)], kv, mask=ind)
                plsc.store_compressed(ti_v.at[pl.ds(cw, 16)], iv, mask=ind)
                return cnt + jnp.sum(ind.astype(jnp.int32))

            c1 = lax.fori_loop(0, nsv, lambda j, c: tsel(j, c, True), 0)
            lax.fori_loop(0, nsv, lambda j, c: tsel(j, c, False), c1)
            v6 = tk_v[pl.ds(96, 16)]
            tk_v[pl.ds(96, 16)] = jnp.where(iota < K - 96, v6, pad16)
            tk_v[pl.ds(112, 16)] = pad16

            ks = [tk_v[pl.ds(t * 16, 16)] for t in range(8)]
            vs = [ti_v[pl.ds(t * 16, 16)] for t in range(8)]
            for t in range(8):
                ks[t], vs[t] = plsc.sort_key_val(ks[t], vs[t], descending=True)
            for (a, bb) in _NET8:
                ks[a], vs[a], ks[bb], vs[bb] = _merge2(ks[a], vs[a], ks[bb], vs[bb])
            for t in range(8):
                tf_v[pl.ds(t * 16, 16)] = _f32_from_key(ks[t])
                ti_v[pl.ds(t * 16, 16)] = vs[t]

            pltpu.sync_copy(tf_v, outv_hbm.at[row])
            pltpu.sync_copy(ti_v, outi_hbm.at[row])
            return _carry

        lax.fori_loop(0, ROWS_PW, row_body, 0)

    return kern(gmax, scores2d)


def kernel(query_ids, query_table, candidate_table, Wq, Wc):
    q_emb = _query_gather(query_table, query_ids)
    cand_p = jnp.pad(candidate_table, ((0, N_PAD - N_CAND), (0, 0)))
    scores, gmax49 = _compute_scores(q_emb, Wq, cand_p, Wc)
    gmax = jnp.transpose(gmax49, (1, 0, 2)).reshape(B, G)
    scores2d = scores.reshape(B * G, S)
    vals, idx = _sc_topk(gmax, scores2d)
    return vals[:, :K], idx[:, :K]


# speedup vs baseline: 1.4447x; 1.4447x over previous
import functools

import jax
import jax.numpy as jnp
from jax import lax
from jax.experimental import pallas as pl
from jax.experimental.pallas import tpu as pltpu
from jax.experimental.pallas import tpu_sc as plsc

B = 1024
D = 64
N_CAND = 100000
K = 100
S = 128
CHUNK = 4096
N_PAD = 102400
N_CHUNKS = N_PAD // CHUNK
G = N_PAD // S
GPC = CHUNK // S
GV = G // 16

NEG = float(jnp.finfo(jnp.float32).min)

NUM_CORES = 2
NUM_SUBCORES = 16
NW = NUM_CORES * NUM_SUBCORES
ROWS_PW = B // NW
GID_CAP = 128
SV = GID_CAP * S // 16
SURV_CAP = 256
SKV = SURV_CAP // 16
OUT_W = 128


def _scores_body(qemb_ref, wq_ref, cand_ref, wc_ref, scores_ref, gmax_ref):
    i = pl.program_id(0)
    q = jnp.dot(qemb_ref[...], wq_ref[...], preferred_element_type=jnp.float32)
    c = jnp.dot(cand_ref[...], wc_ref[...], preferred_element_type=jnp.float32)
    s = lax.dot_general(q, c, (((1,), (1,)), ((), ())),
                        preferred_element_type=jnp.float32)
    col = i * CHUNK + lax.broadcasted_iota(jnp.int32, (B, CHUNK), 1)
    s = jnp.where(col < N_CAND, s, NEG)
    s3 = s.reshape(B, GPC, S)
    scores_ref[...] = s3[None]
    gmax_ref[...] = jnp.max(s3, axis=2)[None]


def _compute_scores(q_emb, Wq, cand_p, Wc):
    return pl.pallas_call(
        _scores_body,
        grid=(N_CHUNKS,),
        in_specs=[
            pl.BlockSpec((B, D), lambda i: (0, 0)),
            pl.BlockSpec((D, D), lambda i: (0, 0)),
            pl.BlockSpec((CHUNK, D), lambda i: (i, 0)),
            pl.BlockSpec((D, D), lambda i: (0, 0)),
        ],
        out_specs=[
            pl.BlockSpec((1, B, GPC, S), lambda i: (i, 0, 0, 0)),
            pl.BlockSpec((1, B, GPC), lambda i: (i, 0, 0)),
        ],
        out_shape=[
            jax.ShapeDtypeStruct((N_CHUNKS, B, GPC, S), jnp.float32),
            jax.ShapeDtypeStruct((N_CHUNKS, B, GPC), jnp.float32),
        ],
    )(q_emb, Wq, cand_p, Wc)


def _key_from_f32(x):
    b = lax.bitcast_convert_type(x, jnp.uint32)
    k = b ^ ((b >> 31) * jnp.uint32(0x7FFFFFFF))
    return lax.bitcast_convert_type(k, jnp.int32)


def _f32_from_key(ki):
    u = lax.bitcast_convert_type(ki, jnp.uint32)
    b = u ^ ((u >> 31) * jnp.uint32(0x7FFFFFFF))
    return lax.bitcast_convert_type(b, jnp.float32)


def _iota16():
    return lax.iota(jnp.int32, 16)


I32_MIN = -2147483648


def _kth_largest(keys_ref, nvec, k_want, nbits, stop_count=None, unroll2=False):
    def count_ge(cand_i):
        if unroll2:
            def cnt_body(j, acc):
                kv0 = keys_ref[pl.ds(j * 32, 16)]
                kv1 = keys_ref[pl.ds(j * 32 + 16, 16)]
                return (acc + (kv0 >= cand_i).astype(jnp.int32)
                        + (kv1 >= cand_i).astype(jnp.int32))
            acc = lax.fori_loop(0, nvec // 2, cnt_body,
                                jnp.zeros((16,), jnp.int32))
        else:
            def cnt_body(j, acc):
                kv = keys_ref[pl.ds(j * 16, 16)]
                return acc + (kv >= cand_i).astype(jnp.int32)
            acc = lax.fori_loop(0, nvec, cnt_body, jnp.zeros((16,), jnp.int32))
        return jnp.sum(acc)

    def step(b, prefix_u):
        shift = (31 - b).astype(jnp.uint32)
        cand_u = prefix_u | (jnp.uint32(1) << shift)
        cand_i = lax.bitcast_convert_type(cand_u ^ jnp.uint32(0x80000000),
                                          jnp.int32)
        cnt = count_ge(cand_i)
        return jnp.where(cnt >= k_want, cand_u, prefix_u), cnt

    if stop_count is None:
        def bit_body(b, prefix_u):
            return step(b, prefix_u)[0]
        prefix_u = lax.fori_loop(0, nbits, bit_body, jnp.uint32(0))
    else:
        def w_cond(st):
            b, _, best_cnt = st
            return (b < nbits) & (best_cnt > stop_count)

        def w_body(st):
            b, prefix_u, best_cnt = st
            new_prefix, cnt = step(b, prefix_u)
            best_cnt = jnp.where(cnt >= k_want, cnt, best_cnt)
            return b + 1, new_prefix, best_cnt

        _, prefix_u, _ = lax.while_loop(
            w_cond, w_body, (jnp.int32(0), jnp.uint32(0), jnp.int32(2**30)))
    return lax.bitcast_convert_type(prefix_u ^ jnp.uint32(0x80000000), jnp.int32)


def _merge2(ka, va, kb, vb):
    kbr = lax.rev(kb, (0,))
    vbr = lax.rev(vb, (0,))
    m = ka >= kbr
    hk = jnp.where(m, ka, kbr)
    hv = jnp.where(m, va, vbr)
    lk = jnp.where(m, kbr, ka)
    lv = jnp.where(m, vbr, va)
    hk, hv = plsc.sort_key_val(hk, hv, descending=True)
    lk, lv = plsc.sort_key_val(lk, lv, descending=True)
    return hk, hv, lk, lv


_NET8 = [(0, 1), (2, 3), (4, 5), (6, 7),
         (0, 2), (1, 3), (4, 6), (5, 7),
         (1, 2), (5, 6),
         (0, 4), (1, 5), (2, 6), (3, 7),
         (2, 4), (3, 5),
         (1, 2), (3, 4), (5, 6)]


def _sc_mesh():
    return plsc.VectorSubcoreMesh(core_axis_name="c", subcore_axis_name="s",
                                  num_cores=NUM_CORES, num_subcores=NUM_SUBCORES)


def _query_gather(query_table, query_ids):
    bpw = B // NW

    @functools.partial(
        pl.kernel, mesh=_sc_mesh(),
        compiler_params=pltpu.CompilerParams(needs_layout_passes=False),
        out_type=jax.ShapeDtypeStruct((B, 128), jnp.float32),
        scratch_types=[pltpu.VMEM((bpw,), jnp.int32),
                       pltpu.VMEM((bpw, 128), jnp.float32),
                       pltpu.SemaphoreType.DMA],
    )
    def kern(table_hbm, ids_hbm, out_hbm, idx_v, rows_v, sem):
        wid = lax.axis_index("s") * NUM_CORES + lax.axis_index("c")
        base = wid * bpw
        pltpu.sync_copy(ids_hbm.at[pl.ds(base, bpw)], idx_v)
        pltpu.async_copy(table_hbm.at[idx_v], rows_v, sem).wait()
        pltpu.sync_copy(rows_v, out_hbm.at[pl.ds(base, bpw)])

    table_p = jnp.pad(query_table, ((0, 0), (0, 128 - D)))
    return kern(table_p, query_ids)[:, :D]


def _sc_topk(gmax, scores2d):
    @functools.partial(
        pl.kernel, mesh=_sc_mesh(),
        compiler_params=pltpu.CompilerParams(needs_layout_passes=False),
        out_type=(jax.ShapeDtypeStruct((B, OUT_W), jnp.float32),
                  jax.ShapeDtypeStruct((B, OUT_W), jnp.int32)),
        scratch_types=[
            pltpu.VMEM((G,), jnp.float32),
            pltpu.VMEM((G,), jnp.int32),
            pltpu.VMEM((GID_CAP + 16,), jnp.int32),
            pltpu.VMEM((GID_CAP,), jnp.int32),
            pltpu.VMEM((GID_CAP, S), jnp.float32),
            pltpu.VMEM((SURV_CAP + 16,), jnp.int32),
            pltpu.VMEM((SURV_CAP + 16,), jnp.int32),
            pltpu.VMEM((OUT_W,), jnp.int32),
            pltpu.VMEM((OUT_W,), jnp.int32),
            pltpu.VMEM((OUT_W,), jnp.float32),
            pltpu.SemaphoreType.DMA,
        ],
    )
    def kern(gmax_hbm, sc2_hbm, outv_hbm, outi_hbm,
             gm_v, gk_v, gid_v, idx_v, gat_v, sk_v, si_v, tk_v, ti_v, tf_v, sem):
        wid = lax.axis_index("s") * NUM_CORES + lax.axis_index("c")
        iota = _iota16()
        zero16 = jnp.zeros((16,), jnp.int32)
        pad16 = jnp.full((16,), I32_MIN, jnp.int32)

        def row_body(rr, _carry):
            row = wid * ROWS_PW + rr

            pltpu.sync_copy(gmax_hbm.at[row], gm_v)

            def keyb(j, c):
                gk_v[pl.ds(j * 16, 16)] = _key_from_f32(gm_v[pl.ds(j * 16, 16)])
                return c
            lax.fori_loop(0, GV, keyb, 0)

            t_g = _kth_largest(gk_v, GV, K, 24,
                               stop_count=GID_CAP, unroll2=True)

            for qq in range(GID_CAP // 16):
                gid_v[pl.ds(qq * 16, 16)] = zero16

            def gsel(j, cnt):
                kv = gk_v[pl.ds(j * 16, 16)]
                ind = kv >= t_g
                cw = jnp.minimum(cnt, GID_CAP)
                plsc.store_compressed(gid_v.at[pl.ds(cw, 16)], j * 16 + iota,
                                      mask=ind)
                return cnt + jnp.sum(ind.astype(jnp.int32))
            cnt_g = jnp.minimum(lax.fori_loop(0, GV, gsel, 0), GID_CAP)

            for qq in range(GID_CAP // 16):
                gv = gid_v[pl.ds(qq * 16, 16)]
                idx_v[pl.ds(qq * 16, 16)] = ((gv >> 5) * (B * GPC)
                                             + row * GPC + (gv & (GPC - 1)))
            pltpu.async_copy(sc2_hbm.at[idx_v], gat_v, sem).wait()

            neg1_16 = zero16 - 1
            for qq in range(SKV):
                sk_v[pl.ds(qq * 16, 16)] = neg1_16
                si_v[pl.ds(qq * 16, 16)] = zero16

            t_gf = _f32_from_key(t_g)

            def fsel(s_slot, scnt):
                gidv = plsc.load_gather(gid_v, [zero16 + s_slot])
                base = gidv * S
                xs = [gat_v[s_slot, pl.ds(q * 16, 16)] for q in range(S // 16)]
                inds = [x >= t_gf for x in xs]
                cnts = [jnp.sum(i.astype(jnp.int32)) for i in inds]
                c = scnt
                for q in range(S // 16):
                    cw = jnp.minimum(c, SURV_CAP)
                    plsc.store_compressed(
                        sk_v.at[pl.ds(cw, 16)],
                        lax.bitcast_convert_type(xs[q], jnp.int32),
                        mask=inds[q])
                    plsc.store_compressed(si_v.at[pl.ds(cw, 16)],
                                          base + q * 16 + iota, mask=inds[q])
                    c = c + cnts[q]
                return c
            scnt = lax.fori_loop(0, cnt_g, fsel, 0)
            scnt_c = jnp.minimum(scnt, SURV_CAP)
            nsv = jnp.minimum((scnt_c + 31) // 32 * 2, SKV)

            def keypass(j, c):
                sk_v[pl.ds(j * 16, 16)] = _key_from_f32(sk_v[pl.ds(j * 16, 16)])
                return c
            lax.fori_loop(0, nsv, keypass, 0)

            t_e = _kth_largest(sk_v, nsv, K, 32, unroll2=True)

            for qq in range(OUT_W // 16):
                tk_v[pl.ds(qq * 16, 16)] = pad16
                ti_v[pl.ds(qq * 16, 16)] = zero16

            def tsel(j, cnt, strict):
                kv = sk_v[pl.ds(j * 16, 16)]
                iv = si_v[pl.ds(j * 16, 16)]
                ind = (kv > t_e) if strict else (kv == t_e)
                cw = jnp.minimum(cnt, K)
                plsc.store_compressed(tk_v.at[pl.ds(cw, 16)], kv, mask=ind)
                plsc.store_compressed(ti_v.at[pl.ds(cw, 16)], iv, mask=ind)
                return cnt + jnp.sum(ind.astype(jnp.int32))

            c1 = lax.fori_loop(0, nsv, lambda j, c: tsel(j, c, True), 0)
            lax.fori_loop(0, nsv, lambda j, c: tsel(j, c, False), c1)
            v6 = tk_v[pl.ds(96, 16)]
            tk_v[pl.ds(96, 16)] = jnp.where(iota < K - 96, v6, pad16)
            tk_v[pl.ds(112, 16)] = pad16

            ks = [tk_v[pl.ds(t * 16, 16)] for t in range(8)]
            vs = [ti_v[pl.ds(t * 16, 16)] for t in range(8)]
            for t in range(8):
                ks[t], vs[t] = plsc.sort_key_val(ks[t], vs[t], descending=True)
            for (a, bb) in _NET8:
                ks[a], vs[a], ks[bb], vs[bb] = _merge2(ks[a], vs[a], ks[bb], vs[bb])
            for t in range(8):
                tf_v[pl.ds(t * 16, 16)] = _f32_from_key(ks[t])
                ti_v[pl.ds(t * 16, 16)] = vs[t]

            pltpu.sync_copy(tf_v, outv_hbm.at[row])
            pltpu.sync_copy(ti_v, outi_hbm.at[row])
            return _carry

        lax.fori_loop(0, ROWS_PW, row_body, 0)

    return kern(gmax, scores2d)


def kernel(query_ids, query_table, candidate_table, Wq, Wc):
    q_emb = _query_gather(query_table, query_ids)
    cand_p = jnp.pad(candidate_table, ((0, N_PAD - N_CAND), (0, 0)))
    scores, gmax49 = _compute_scores(q_emb, Wq, cand_p, Wc)
    gmax = jnp.transpose(gmax49, (1, 0, 2)).reshape(B, G)
    scores2d = scores.reshape(N_CHUNKS * B * GPC, S)
    vals, idx = _sc_topk(gmax, scores2d)
    return vals[:, :K], idx[:, :K]


# speedup vs baseline: 1.4555x; 1.0074x over previous
import functools

import jax
import jax.numpy as jnp
from jax import lax
from jax.experimental import pallas as pl
from jax.experimental.pallas import tpu as pltpu
from jax.experimental.pallas import tpu_sc as plsc

B = 1024
D = 64
N_CAND = 100000
K = 100
S = 128
CHUNK = 4096
N_PAD = 102400
N_CHUNKS = N_PAD // CHUNK
G = N_PAD // S
GPC = CHUNK // S
GV = G // 16

NEG = float(jnp.finfo(jnp.float32).min)

NUM_CORES = 2
NUM_SUBCORES = 16
NW = NUM_CORES * NUM_SUBCORES
ROWS_PW = B // NW
GID_CAP = 128
SV = GID_CAP * S // 16
SURV_CAP = 256
SKV = SURV_CAP // 16
OUT_W = 128


def _scores_body(qemb_ref, wq_ref, cand_ref, wc_ref, scores_ref, gmax_ref):
    i = pl.program_id(0)
    q = jnp.dot(qemb_ref[...], wq_ref[...], preferred_element_type=jnp.float32)
    c = jnp.dot(cand_ref[...], wc_ref[...], preferred_element_type=jnp.float32)
    s = lax.dot_general(q, c, (((1,), (1,)), ((), ())),
                        preferred_element_type=jnp.float32)
    col = i * CHUNK + lax.broadcasted_iota(jnp.int32, (B, CHUNK), 1)
    s = jnp.where(col < N_CAND, s, NEG)
    s3 = s.reshape(B, GPC, S)
    scores_ref[...] = s3[None]
    gmax_ref[...] = jnp.max(s3, axis=2)[None]


def _compute_scores(q_emb, Wq, cand_p, Wc):
    return pl.pallas_call(
        _scores_body,
        grid=(N_CHUNKS,),
        in_specs=[
            pl.BlockSpec((B, D), lambda i: (0, 0)),
            pl.BlockSpec((D, D), lambda i: (0, 0)),
            pl.BlockSpec((CHUNK, D), lambda i: (i, 0)),
            pl.BlockSpec((D, D), lambda i: (0, 0)),
        ],
        out_specs=[
            pl.BlockSpec((1, B, GPC, S), lambda i: (i, 0, 0, 0)),
            pl.BlockSpec((1, B, GPC), lambda i: (i, 0, 0)),
        ],
        out_shape=[
            jax.ShapeDtypeStruct((N_CHUNKS, B, GPC, S), jnp.float32),
            jax.ShapeDtypeStruct((N_CHUNKS, B, GPC), jnp.float32),
        ],
    )(q_emb, Wq, cand_p, Wc)


def _key_from_f32(x):
    b = lax.bitcast_convert_type(x, jnp.uint32)
    k = b ^ ((b >> 31) * jnp.uint32(0x7FFFFFFF))
    return lax.bitcast_convert_type(k, jnp.int32)


def _f32_from_key(ki):
    u = lax.bitcast_convert_type(ki, jnp.uint32)
    b = u ^ ((u >> 31) * jnp.uint32(0x7FFFFFFF))
    return lax.bitcast_convert_type(b, jnp.float32)


def _iota16():
    return lax.iota(jnp.int32, 16)


I32_MIN = -2147483648


def _kth_largest(keys_ref, nvec, k_want, nbits, stop_count=None, unroll2=False):
    def count_ge(cand_i):
        if unroll2:
            def cnt_body(j, acc):
                kv0 = keys_ref[pl.ds(j * 32, 16)]
                kv1 = keys_ref[pl.ds(j * 32 + 16, 16)]
                return (acc + (kv0 >= cand_i).astype(jnp.int32)
                        + (kv1 >= cand_i).astype(jnp.int32))
            acc = lax.fori_loop(0, nvec // 2, cnt_body,
                                jnp.zeros((16,), jnp.int32))
        else:
            def cnt_body(j, acc):
                kv = keys_ref[pl.ds(j * 16, 16)]
                return acc + (kv >= cand_i).astype(jnp.int32)
            acc = lax.fori_loop(0, nvec, cnt_body, jnp.zeros((16,), jnp.int32))
        return jnp.sum(acc)

    def step(b, prefix_u):
        shift = (31 - b).astype(jnp.uint32)
        cand_u = prefix_u | (jnp.uint32(1) << shift)
        cand_i = lax.bitcast_convert_type(cand_u ^ jnp.uint32(0x80000000),
                                          jnp.int32)
        cnt = count_ge(cand_i)
        return jnp.where(cnt >= k_want, cand_u, prefix_u), cnt

    if stop_count is None:
        def bit_body(b, prefix_u):
            return step(b, prefix_u)[0]
        prefix_u = lax.fori_loop(0, nbits, bit_body, jnp.uint32(0))
    else:
        def w_cond(st):
            b, _, best_cnt = st
            return (b < nbits) & (best_cnt > stop_count)

        def w_body(st):
            b, prefix_u, best_cnt = st
            new_prefix, cnt = step(b, prefix_u)
            best_cnt = jnp.where(cnt >= k_want, cnt, best_cnt)
            return b + 1, new_prefix, best_cnt

        _, prefix_u, _ = lax.while_loop(
            w_cond, w_body, (jnp.int32(0), jnp.uint32(0), jnp.int32(2**30)))
    return lax.bitcast_convert_type(prefix_u ^ jnp.uint32(0x80000000), jnp.int32)


def _merge2(ka, va, kb, vb):
    kbr = lax.rev(kb, (0,))
    vbr = lax.rev(vb, (0,))
    m = ka >= kbr
    hk = jnp.where(m, ka, kbr)
    hv = jnp.where(m, va, vbr)
    lk = jnp.where(m, kbr, ka)
    lv = jnp.where(m, vbr, va)
    hk, hv = plsc.sort_key_val(hk, hv, descending=True)
    lk, lv = plsc.sort_key_val(lk, lv, descending=True)
    return hk, hv, lk, lv


_NET8 = [(0, 1), (2, 3), (4, 5), (6, 7),
         (0, 2), (1, 3), (4, 6), (5, 7),
         (1, 2), (5, 6),
         (0, 4), (1, 5), (2, 6), (3, 7),
         (2, 4), (3, 5),
         (1, 2), (3, 4), (5, 6)]


def _sc_mesh():
    return plsc.VectorSubcoreMesh(core_axis_name="c", subcore_axis_name="s",
                                  num_cores=NUM_CORES, num_subcores=NUM_SUBCORES)


def _query_gather(query_table, query_ids):
    bpw = B // NW

    @functools.partial(
        pl.kernel, mesh=_sc_mesh(),
        compiler_params=pltpu.CompilerParams(needs_layout_passes=False),
        out_type=jax.ShapeDtypeStruct((B, 128), jnp.float32),
        scratch_types=[pltpu.VMEM((bpw,), jnp.int32),
                       pltpu.VMEM((bpw, 128), jnp.float32),
                       pltpu.SemaphoreType.DMA],
    )
    def kern(table_hbm, ids_hbm, out_hbm, idx_v, rows_v, sem):
        wid = lax.axis_index("s") * NUM_CORES + lax.axis_index("c")
        base = wid * bpw
        pltpu.sync_copy(ids_hbm.at[pl.ds(base, bpw)], idx_v)
        pltpu.async_copy(table_hbm.at[idx_v], rows_v, sem).wait()
        pltpu.sync_copy(rows_v, out_hbm.at[pl.ds(base, bpw)])

    table_p = jnp.pad(query_table, ((0, 0), (0, 128 - D)))
    return kern(table_p, query_ids)[:, :D]


def _sc_topk(gmax3, scores2d):
    @functools.partial(
        pl.kernel, mesh=_sc_mesh(),
        compiler_params=pltpu.CompilerParams(needs_layout_passes=False),
        out_type=(jax.ShapeDtypeStruct((B, OUT_W), jnp.float32),
                  jax.ShapeDtypeStruct((B, OUT_W), jnp.int32)),
        scratch_types=[
            pltpu.VMEM((N_CHUNKS, GPC), jnp.float32),
            pltpu.VMEM((G,), jnp.int32),
            pltpu.VMEM((GID_CAP + 16,), jnp.int32),
            pltpu.VMEM((GID_CAP,), jnp.int32),
            pltpu.VMEM((GID_CAP, S), jnp.float32),
            pltpu.VMEM((SURV_CAP + 16,), jnp.int32),
            pltpu.VMEM((SURV_CAP + 16,), jnp.int32),
            pltpu.VMEM((OUT_W,), jnp.int32),
            pltpu.VMEM((OUT_W,), jnp.int32),
            pltpu.VMEM((OUT_W,), jnp.float32),
            pltpu.SemaphoreType.DMA,
        ],
    )
    def kern(gmax_hbm, sc2_hbm, outv_hbm, outi_hbm,
             gm_v, gk_v, gid_v, idx_v, gat_v, sk_v, si_v, tk_v, ti_v, tf_v, sem):
        wid = lax.axis_index("s") * NUM_CORES + lax.axis_index("c")
        iota = _iota16()
        zero16 = jnp.zeros((16,), jnp.int32)
        pad16 = jnp.full((16,), I32_MIN, jnp.int32)

        def row_body(rr, _carry):
            row = wid * ROWS_PW + rr

            pltpu.sync_copy(gmax_hbm.at[:, row, :], gm_v)

            def keyb(j, c):
                gk_v[pl.ds(j * 16, 16)] = _key_from_f32(
                    gm_v[j >> 1, pl.ds((j & 1) * 16, 16)])
                return c
            lax.fori_loop(0, GV, keyb, 0)

            t_g = _kth_largest(gk_v, GV, K, 24,
                               stop_count=GID_CAP, unroll2=True)

            for qq in range(GID_CAP // 16):
                gid_v[pl.ds(qq * 16, 16)] = zero16

            def gsel(j, cnt):
                kv = gk_v[pl.ds(j * 16, 16)]
                ind = kv >= t_g
                cw = jnp.minimum(cnt, GID_CAP)
                plsc.store_compressed(gid_v.at[pl.ds(cw, 16)], j * 16 + iota,
                                      mask=ind)
                return cnt + jnp.sum(ind.astype(jnp.int32))
            cnt_g = jnp.minimum(lax.fori_loop(0, GV, gsel, 0), GID_CAP)

            for qq in range(GID_CAP // 16):
                gv = gid_v[pl.ds(qq * 16, 16)]
                idx_v[pl.ds(qq * 16, 16)] = ((gv >> 5) * (B * GPC)
                                             + row * GPC + (gv & (GPC - 1)))
            pltpu.async_copy(sc2_hbm.at[idx_v], gat_v, sem).wait()

            neg1_16 = zero16 - 1
            for qq in range(SKV):
                sk_v[pl.ds(qq * 16, 16)] = neg1_16
                si_v[pl.ds(qq * 16, 16)] = zero16

            t_gf = _f32_from_key(t_g)

            def fsel(s_slot, scnt):
                gidv = plsc.load_gather(gid_v, [zero16 + s_slot])
                base = gidv * S
                xs = [gat_v[s_slot, pl.ds(q * 16, 16)] for q in range(S // 16)]
                inds = [x >= t_gf for x in xs]
                cnts = [jnp.sum(i.astype(jnp.int32)) for i in inds]
                c = scnt
                for q in range(S // 16):
                    cw = jnp.minimum(c, SURV_CAP)
                    plsc.store_compressed(
                        sk_v.at[pl.ds(cw, 16)],
                        lax.bitcast_convert_type(xs[q], jnp.int32),
                        mask=inds[q])
                    plsc.store_compressed(si_v.at[pl.ds(cw, 16)],
                                          base + q * 16 + iota, mask=inds[q])
                    c = c + cnts[q]
                return c
            scnt = lax.fori_loop(0, cnt_g, fsel, 0)
            scnt_c = jnp.minimum(scnt, SURV_CAP)
            nsv = jnp.minimum((scnt_c + 31) // 32 * 2, SKV)

            def keypass(j, c):
                sk_v[pl.ds(j * 16, 16)] = _key_from_f32(sk_v[pl.ds(j * 16, 16)])
                return c
            lax.fori_loop(0, nsv, keypass, 0)

            t_e = _kth_largest(sk_v, nsv, K, 32, unroll2=True)

            for qq in range(OUT_W // 16):
                tk_v[pl.ds(qq * 16, 16)] = pad16
                ti_v[pl.ds(qq * 16, 16)] = zero16

            def tsel(j, cnt, strict):
                kv = sk_v[pl.ds(j * 16, 16)]
                iv = si_v[pl.ds(j * 16, 16)]
                ind = (kv > t_e) if strict else (kv == t_e)
                cw = jnp.minimum(cnt, K)
                plsc.store_compressed(tk_v.at[pl.ds(cw, 16)], kv, mask=ind)
                plsc.store_compressed(ti_v.at[pl.ds(cw, 16)], iv, mask=ind)
                return cnt + jnp.sum(ind.astype(jnp.int32))

            c1 = lax.fori_loop(0, nsv, lambda j, c: tsel(j, c, True), 0)
            lax.fori_loop(0, nsv, lambda j, c: tsel(j, c, False), c1)
            v6 = tk_v[pl.ds(96, 16)]
            tk_v[pl.ds(96, 16)] = jnp.where(iota < K - 96, v6, pad16)
            tk_v[pl.ds(112, 16)] = pad16

            ks = [tk_v[pl.ds(t * 16, 16)] for t in range(8)]
            vs = [ti_v[pl.ds(t * 16, 16)] for t in range(8)]
            for t in range(8):
                ks[t], vs[t] = plsc.sort_key_val(ks[t], vs[t], descending=True)
            for (a, bb) in _NET8:
                ks[a], vs[a], ks[bb], vs[bb] = _merge2(ks[a], vs[a], ks[bb], vs[bb])
            for t in range(8):
                tf_v[pl.ds(t * 16, 16)] = _f32_from_key(ks[t])
                ti_v[pl.ds(t * 16, 16)] = vs[t]

            pltpu.sync_copy(tf_v, outv_hbm.at[row])
            pltpu.sync_copy(ti_v, outi_hbm.at[row])
            return _carry

        lax.fori_loop(0, ROWS_PW, row_body, 0)

    return kern(gmax3, scores2d)


def kernel(query_ids, query_table, candidate_table, Wq, Wc):
    q_emb = _query_gather(query_table, query_ids)
    cand_p = jnp.pad(candidate_table, ((0, N_PAD - N_CAND), (0, 0)))
    scores, gmax49 = _compute_scores(q_emb, Wq, cand_p, Wc)
    scores2d = scores.reshape(N_CHUNKS * B * GPC, S)
    vals, idx = _sc_topk(gmax49, scores2d)
    return vals[:, :K], idx[:, :K]
